# baseline (speedup 1.0000x reference)
import functools

import jax
import jax.numpy as jnp
from jax import lax
from jax.experimental import pallas as pl
from jax.experimental.pallas import tpu as pltpu
from jax.experimental.pallas import tpu_sc as plsc

N_NODES = 10000
N_EDGES = 320000
IN_CH = 128
OUT_CH = 128
HID = 256

N_TILES = 16
N_CORES = 2
CHUNK = 128
N_PAD = 10112
E_PAD = 323584
ROWS_PER_TILE = N_PAD // N_TILES


def _sc_mesh():
    return plsc.VectorSubcoreMesh(core_axis_name="c", subcore_axis_name="s")


DEG_W = 128
CH_SPLIT = E_PAD // N_TILES // CHUNK
CH_EDGE = E_PAD // (N_CORES * N_TILES) // CHUNK


def _deg_body(dst_h, ones_h, zeros_h, out_h, dst_v, ones_v, deg_sp, ssem):
    c = lax.axis_index("c")
    s = lax.axis_index("s")
    r0 = s * ROWS_PER_TILE
    pltpu.sync_copy(zeros_h.at[pl.ds(r0, ROWS_PER_TILE)],
                    deg_sp.at[pl.ds(r0, ROWS_PER_TILE)])
    pltpu.sync_copy(ones_h, ones_v)
    pltpu.sync_copy(dst_h.at[c * N_TILES + s], dst_v)
    plsc.subcore_barrier()

    def scat(i, b):
        return pltpu.make_async_copy(ones_v, deg_sp.at[dst_v.at[i]],
                                     ssem.at[b])

    def chunk(i, carry):
        b = lax.rem(i, 2)

        @pl.when(i >= 2)
        def _():
            scat(i - 2, b).wait()

        scat(i, b).start(add=True)
        return carry

    lax.fori_loop(0, CH_EDGE, chunk, 0)
    scat(CH_EDGE - 2, lax.rem(CH_EDGE - 2, 2)).wait()
    scat(CH_EDGE - 1, lax.rem(CH_EDGE - 1, 2)).wait()
    plsc.subcore_barrier()
    pltpu.sync_copy(deg_sp.at[pl.ds(r0, ROWS_PER_TILE)],
                    out_h.at[c, pl.ds(r0, ROWS_PER_TILE)])


_deg_call = pl.kernel(
    _deg_body,
    out_type=jax.ShapeDtypeStruct((N_CORES, N_PAD, DEG_W), jnp.float32),
    mesh=_sc_mesh(),
    scratch_types=[
        pltpu.VMEM((CH_EDGE, CHUNK), jnp.int32),
        pltpu.VMEM((CHUNK, DEG_W), jnp.float32),
        pltpu.VMEM_SHARED((N_PAD, DEG_W), jnp.float32),
        pltpu.SemaphoreType.DMA((2,)),
    ],
)


def _make_agg(feature_split):
    n_chunks = CH_SPLIT if feature_split else CH_EDGE
    NR = 2
    ND = 4

    def body(table_h, src_h, dst_h, zeros_h, out_h, idx_s, idx_d, rows_v,
             agg_sp, isem, dsem, gsem, ssem):
        c = lax.axis_index("c")
        s = lax.axis_index("s")
        r0 = s * ROWS_PER_TILE
        pltpu.sync_copy(zeros_h.at[pl.ds(r0, ROWS_PER_TILE)],
                        agg_sp.at[pl.ds(r0, ROWS_PER_TILE)])
        plsc.subcore_barrier()

        def isrc(i):
            b = lax.rem(i, NR)
            if feature_split:
                src_row = src_h.at[c, s, i]
            else:
                src_row = src_h.at[c * N_TILES + s, i]
            return pltpu.make_async_copy(src_row, idx_s.at[b], isem.at[b])

        def idst(i):
            b = lax.rem(i, ND)
            if feature_split:
                dst_row = dst_h.at[s, i]
            else:
                dst_row = dst_h.at[c * N_TILES + s, i]
            return pltpu.make_async_copy(dst_row, idx_d.at[b], dsem.at[b])

        def gath(i):
            b = lax.rem(i, NR)
            return pltpu.make_async_copy(table_h.at[idx_s.at[b]],
                                         rows_v.at[b], gsem.at[b])

        def scat(i):
            b = lax.rem(i, NR)
            return pltpu.make_async_copy(rows_v.at[b],
                                         agg_sp.at[idx_d.at[lax.rem(i, ND)]],
                                         ssem.at[b])

        isrc(0).start()
        idst(0).start()
        isrc(1).start()
        idst(1).start()
        isrc(0).wait()
        idst(0).wait()
        gath(0).start()

        def chunk(i, carry):
            @pl.when(i >= 1)
            def _():
                scat(i - 1).wait()

            gath(i).wait()
            scat(i).start(add=True)

            @pl.when(i + 1 < n_chunks)
            def _():
                isrc(i + 1).wait()
                idst(i + 1).wait()
                gath(i + 1).start()

            @pl.when(i + 2 < n_chunks)
            def _():
                isrc(i + 2).start()
                idst(i + 2).start()

            return carry

        lax.fori_loop(0, n_chunks, chunk, 0)
        scat(n_chunks - 1).wait()
        plsc.subcore_barrier()
        pltpu.sync_copy(agg_sp.at[pl.ds(r0, ROWS_PER_TILE)],
                        out_h.at[c, pl.ds(r0, ROWS_PER_TILE)])

    return pl.kernel(
        body,
        out_type=jax.ShapeDtypeStruct((N_CORES, N_PAD, OUT_CH), jnp.float32),
        mesh=_sc_mesh(),
        scratch_types=[
            pltpu.VMEM((NR, CHUNK), jnp.int32),
            pltpu.VMEM((ND, CHUNK), jnp.int32),
            pltpu.VMEM((NR, CHUNK, OUT_CH), jnp.float32),
            pltpu.VMEM_SHARED((N_PAD, OUT_CH), jnp.float32),
            pltpu.SemaphoreType.DMA((NR,)),
            pltpu.SemaphoreType.DMA((ND,)),
            pltpu.SemaphoreType.DMA((NR,)),
            pltpu.SemaphoreType.DMA((NR,)),
        ],
    )


_agg_feat_call = _make_agg(feature_split=True)
_agg_edge_call = _make_agg(feature_split=False)


def _mm1_body(x_ref, w_ref, degp_ref, hs_ref, dinv_ref):
    deg = degp_ref[0] + degp_ref[1] + 1.0
    dinv = jax.lax.rsqrt(deg)
    h = jnp.dot(x_ref[...], w_ref[...], preferred_element_type=jnp.float32)
    hs = h * dinv
    hs_ref[0] = hs[:, :OUT_CH]
    hs_ref[1] = hs[:, OUT_CH:]
    dinv_ref[...] = dinv


def _mm2_body(agg_ref, hs1_ref, dinv_ref, b1_ref, w2_ref, hs2_ref):
    dinv = dinv_ref[...]
    xa = jnp.maximum((agg_ref[0] + hs1_ref[0]) * dinv + b1_ref[:, :OUT_CH], 0.0)
    xb = jnp.maximum((agg_ref[1] + hs1_ref[1]) * dinv + b1_ref[:, OUT_CH:], 0.0)
    w2 = w2_ref[...]
    h2 = (jnp.dot(xa, w2[:OUT_CH], preferred_element_type=jnp.float32)
          + jnp.dot(xb, w2[OUT_CH:], preferred_element_type=jnp.float32))
    hs2 = h2 * dinv
    hs2_ref[0] = hs2
    hs2_ref[1] = hs2


def _ep2_body(agg_ref, hs2_ref, dinv_ref, b2_ref, o_ref):
    acc = agg_ref[0] + agg_ref[1] + hs2_ref[0]
    o_ref[...] = jnp.maximum(acc * dinv_ref[...] + b2_ref[...], 0.0)


_TC_R = 1264
_GRID = (N_PAD // _TC_R,)


def _mm1_call(x_pad, W1, degp_col):
    return pl.pallas_call(
        _mm1_body,
        grid=_GRID,
        in_specs=[
            pl.BlockSpec((_TC_R, IN_CH), lambda i: (i, 0)),
            pl.BlockSpec((IN_CH, HID), lambda i: (0, 0)),
            pl.BlockSpec((2, _TC_R, 1), lambda i: (0, i, 0)),
        ],
        out_specs=[
            pl.BlockSpec((2, _TC_R, OUT_CH), lambda i: (0, i, 0)),
            pl.BlockSpec((_TC_R, 1), lambda i: (i, 0)),
        ],
        out_shape=[
            jax.ShapeDtypeStruct((2, N_PAD, OUT_CH), jnp.float32),
            jax.ShapeDtypeStruct((N_PAD, 1), jnp.float32),
        ],
    )(x_pad, W1, degp_col)


def _mm2_call(agg1, hs1, dinv, b1, W2):
    return pl.pallas_call(
        _mm2_body,
        grid=_GRID,
        in_specs=[
            pl.BlockSpec((2, _TC_R, OUT_CH), lambda i: (0, i, 0)),
            pl.BlockSpec((2, _TC_R, OUT_CH), lambda i: (0, i, 0)),
            pl.BlockSpec((_TC_R, 1), lambda i: (i, 0)),
            pl.BlockSpec((1, HID), lambda i: (0, 0)),
            pl.BlockSpec((HID, OUT_CH), lambda i: (0, 0)),
        ],
        out_specs=pl.BlockSpec((2, _TC_R, OUT_CH), lambda i: (0, i, 0)),
        out_shape=jax.ShapeDtypeStruct((2, N_PAD, OUT_CH), jnp.float32),
    )(agg1, hs1, dinv, b1, W2)


def _ep2_call(agg2, hs2d, dinv, b2):
    return pl.pallas_call(
        _ep2_body,
        grid=_GRID,
        in_specs=[
            pl.BlockSpec((2, _TC_R, OUT_CH), lambda i: (0, i, 0)),
            pl.BlockSpec((1, _TC_R, OUT_CH), lambda i: (0, i, 0)),
            pl.BlockSpec((_TC_R, 1), lambda i: (i, 0)),
            pl.BlockSpec((1, OUT_CH), lambda i: (0, 0)),
        ],
        out_specs=pl.BlockSpec((_TC_R, OUT_CH), lambda i: (i, 0)),
        out_shape=jax.ShapeDtypeStruct((N_PAD, OUT_CH), jnp.float32),
    )(agg2, hs2d, dinv, b2)


@jax.jit
def kernel(x, edge_index, W1, b1, W2, b2):
    src = edge_index[0].astype(jnp.int32)
    dst = edge_index[1].astype(jnp.int32)
    pad = E_PAD - N_EDGES
    fill_src = jnp.full((pad,), N_NODES, jnp.int32)
    fill_dst = N_NODES + (jnp.arange(pad, dtype=jnp.int32) % (N_PAD - N_NODES))
    src_p = jnp.concatenate([src, fill_src])
    dst_p = jnp.concatenate([dst, fill_dst])
    src3 = src_p.reshape(N_TILES, CH_SPLIT, CHUNK)
    src4 = jnp.stack([src3, src3 + N_PAD])
    dst3 = dst_p.reshape(N_TILES, CH_SPLIT, CHUNK)
    srcr = src_p.reshape(N_CORES * N_TILES, CH_EDGE, CHUNK)
    srcr_off = jnp.concatenate([srcr[:N_TILES], srcr[N_TILES:] + N_PAD])
    dstr = dst_p.reshape(N_CORES * N_TILES, CH_EDGE, CHUNK)
    x_pad = jnp.pad(x, ((0, N_PAD - N_NODES), (0, 0)))
    zeros_nd = jnp.zeros((N_PAD, OUT_CH), jnp.float32)
    ones_c1 = jnp.ones((CHUNK, DEG_W), jnp.float32)

    degp = _deg_call(dstr, ones_c1, zeros_nd)
    degp_col = degp[:, :, :1]
    hs1, dinv = _mm1_call(x_pad, W1, degp_col)
    agg1 = _agg_feat_call(hs1.reshape(2 * N_PAD, OUT_CH), src4, dst3,
                          zeros_nd)
    hs2d = _mm2_call(agg1, hs1, dinv, b1.reshape(1, HID), W2)
    agg2 = _agg_edge_call(hs2d.reshape(2 * N_PAD, OUT_CH), srcr_off, dstr,
                          zeros_nd)
    out = _ep2_call(agg2, hs2d, dinv, b2.reshape(1, OUT_CH))
    return out[:N_NODES]


# speedup vs baseline: 1.1209x; 1.1209x over previous
import functools

import jax
import jax.numpy as jnp
from jax import lax
from jax.experimental import pallas as pl
from jax.experimental.pallas import tpu as pltpu
from jax.experimental.pallas import tpu_sc as plsc

N_NODES = 10000
N_EDGES = 320000
IN_CH = 128
OUT_CH = 128
HID = 256

N_TILES = 16
N_CORES = 2
CHUNK = 128
N_PAD = 10112
E_PAD = 323584
ROWS_PER_TILE = N_PAD // N_TILES


def _sc_mesh():
    return plsc.VectorSubcoreMesh(core_axis_name="c", subcore_axis_name="s")


DEG_W = 128
CH_EDGE = E_PAD // (N_CORES * N_TILES) // CHUNK
CHUNK_A = 64
CHA_SPLIT = E_PAD // N_TILES // CHUNK_A
CHA_EDGE = E_PAD // (N_CORES * N_TILES) // CHUNK_A


def _deg_body(dst_h, ones_h, zeros_h, out_h, dst_v, ones_v, deg_sp, ssem):
    c = lax.axis_index("c")
    s = lax.axis_index("s")
    r0 = s * ROWS_PER_TILE
    pltpu.sync_copy(zeros_h.at[pl.ds(r0, ROWS_PER_TILE)],
                    deg_sp.at[pl.ds(r0, ROWS_PER_TILE)])
    pltpu.sync_copy(ones_h, ones_v)
    pltpu.sync_copy(dst_h.at[c * N_TILES + s], dst_v)
    plsc.subcore_barrier()

    def scat(i, b):
        return pltpu.make_async_copy(ones_v, deg_sp.at[dst_v.at[i]],
                                     ssem.at[b])

    def chunk(i, carry):
        b = lax.rem(i, 2)

        @pl.when(i >= 2)
        def _():
            scat(i - 2, b).wait()

        scat(i, b).start(add=True)
        return carry

    lax.fori_loop(0, CH_EDGE, chunk, 0)
    scat(CH_EDGE - 2, lax.rem(CH_EDGE - 2, 2)).wait()
    scat(CH_EDGE - 1, lax.rem(CH_EDGE - 1, 2)).wait()
    plsc.subcore_barrier()
    pltpu.sync_copy(deg_sp.at[pl.ds(r0, ROWS_PER_TILE)],
                    out_h.at[c, pl.ds(r0, ROWS_PER_TILE)])


_deg_call = pl.kernel(
    _deg_body,
    out_type=jax.ShapeDtypeStruct((N_CORES, N_PAD, DEG_W), jnp.float32),
    mesh=_sc_mesh(),
    scratch_types=[
        pltpu.VMEM((CH_EDGE, CHUNK), jnp.int32),
        pltpu.VMEM((CHUNK, DEG_W), jnp.float32),
        pltpu.VMEM_SHARED((N_PAD, DEG_W), jnp.float32),
        pltpu.SemaphoreType.DMA((2,)),
    ],
)


def _make_agg(feature_split):
    n_chunks = CHA_SPLIT if feature_split else CHA_EDGE
    NR = 5
    G = 3
    NI = 8

    def body(table_h, src_h, dst_h, zeros_h, out_h, idx_s, idx_d, rows_v,
             agg_sp, isem, dsem, gsem, ssem):
        c = lax.axis_index("c")
        s = lax.axis_index("s")
        r0 = s * ROWS_PER_TILE
        pltpu.sync_copy(zeros_h.at[pl.ds(r0, ROWS_PER_TILE)],
                        agg_sp.at[pl.ds(r0, ROWS_PER_TILE)])
        plsc.subcore_barrier()

        def isrc(i):
            b = lax.rem(i, NI)
            if feature_split:
                src_row = src_h.at[c, s, i]
            else:
                src_row = src_h.at[c * N_TILES + s, i]
            return pltpu.make_async_copy(src_row, idx_s.at[b], isem.at[b])

        def idst(i):
            b = lax.rem(i, NI)
            if feature_split:
                dst_row = dst_h.at[s, i]
            else:
                dst_row = dst_h.at[c * N_TILES + s, i]
            return pltpu.make_async_copy(dst_row, idx_d.at[b], dsem.at[b])

        def gath(i):
            b = lax.rem(i, NR)
            return pltpu.make_async_copy(table_h.at[idx_s.at[lax.rem(i, NI)]],
                                         rows_v.at[b], gsem.at[b])

        def scat(i):
            b = lax.rem(i, NR)
            return pltpu.make_async_copy(rows_v.at[b],
                                         agg_sp.at[idx_d.at[lax.rem(i, NI)]],
                                         ssem.at[b])

        for k in range(G + 1):
            isrc(k).start()
            idst(k).start()
        for k in range(G):
            isrc(k).wait()
            idst(k).wait()
            gath(k).start()

        def chunk(i, carry):
            @pl.when(i >= NR - G)
            def _():
                scat(i - (NR - G)).wait()

            gath(i).wait()
            scat(i).start(add=True)

            @pl.when(i + G < n_chunks)
            def _():
                isrc(i + G).wait()
                idst(i + G).wait()
                gath(i + G).start()

            @pl.when(i + G + 1 < n_chunks)
            def _():
                isrc(i + G + 1).start()
                idst(i + G + 1).start()

            return carry

        lax.fori_loop(0, n_chunks, chunk, 0)
        for k in range(NR - G):
            scat(n_chunks - (NR - G) + k).wait()
        plsc.subcore_barrier()
        pltpu.sync_copy(agg_sp.at[pl.ds(r0, ROWS_PER_TILE)],
                        out_h.at[c, pl.ds(r0, ROWS_PER_TILE)])

    return pl.kernel(
        body,
        out_type=jax.ShapeDtypeStruct((N_CORES, N_PAD, OUT_CH), jnp.float32),
        mesh=_sc_mesh(),
        scratch_types=[
            pltpu.VMEM((NI, CHUNK_A), jnp.int32),
            pltpu.VMEM((NI, CHUNK_A), jnp.int32),
            pltpu.VMEM((NR, CHUNK_A, OUT_CH), jnp.float32),
            pltpu.VMEM_SHARED((N_PAD, OUT_CH), jnp.float32),
            pltpu.SemaphoreType.DMA((NI,)),
            pltpu.SemaphoreType.DMA((NI,)),
            pltpu.SemaphoreType.DMA((NR,)),
            pltpu.SemaphoreType.DMA((NR,)),
        ],
    )


_agg_feat_call = _make_agg(feature_split=True)
_agg_edge_call = _make_agg(feature_split=False)


def _mm1_body(x_ref, w_ref, degp_ref, hs_ref, dinv_ref):
    deg = degp_ref[0] + degp_ref[1] + 1.0
    dinv = jax.lax.rsqrt(deg)
    h = jnp.dot(x_ref[...], w_ref[...], preferred_element_type=jnp.float32)
    hs = h * dinv
    hs_ref[0] = hs[:, :OUT_CH]
    hs_ref[1] = hs[:, OUT_CH:]
    dinv_ref[...] = dinv


def _mm2_body(agg_ref, hs1_ref, dinv_ref, b1_ref, w2_ref, hs2_ref):
    dinv = dinv_ref[...]
    xa = jnp.maximum((agg_ref[0] + hs1_ref[0]) * dinv + b1_ref[:, :OUT_CH], 0.0)
    xb = jnp.maximum((agg_ref[1] + hs1_ref[1]) * dinv + b1_ref[:, OUT_CH:], 0.0)
    w2 = w2_ref[...]
    h2 = (jnp.dot(xa, w2[:OUT_CH], preferred_element_type=jnp.float32)
          + jnp.dot(xb, w2[OUT_CH:], preferred_element_type=jnp.float32))
    hs2 = h2 * dinv
    hs2_ref[0] = hs2
    hs2_ref[1] = hs2


def _ep2_body(agg_ref, hs2_ref, dinv_ref, b2_ref, o_ref):
    acc = agg_ref[0] + agg_ref[1] + hs2_ref[0]
    o_ref[...] = jnp.maximum(acc * dinv_ref[...] + b2_ref[...], 0.0)


_TC_R = 1264
_GRID = (N_PAD // _TC_R,)


def _mm1_call(x_pad, W1, degp_col):
    return pl.pallas_call(
        _mm1_body,
        grid=_GRID,
        in_specs=[
            pl.BlockSpec((_TC_R, IN_CH), lambda i: (i, 0)),
            pl.BlockSpec((IN_CH, HID), lambda i: (0, 0)),
            pl.BlockSpec((2, _TC_R, 1), lambda i: (0, i, 0)),
        ],
        out_specs=[
            pl.BlockSpec((2, _TC_R, OUT_CH), lambda i: (0, i, 0)),
            pl.BlockSpec((_TC_R, 1), lambda i: (i, 0)),
        ],
        out_shape=[
            jax.ShapeDtypeStruct((2, N_PAD, OUT_CH), jnp.float32),
            jax.ShapeDtypeStruct((N_PAD, 1), jnp.float32),
        ],
    )(x_pad, W1, degp_col)


def _mm2_call(agg1, hs1, dinv, b1, W2):
    return pl.pallas_call(
        _mm2_body,
        grid=_GRID,
        in_specs=[
            pl.BlockSpec((2, _TC_R, OUT_CH), lambda i: (0, i, 0)),
            pl.BlockSpec((2, _TC_R, OUT_CH), lambda i: (0, i, 0)),
            pl.BlockSpec((_TC_R, 1), lambda i: (i, 0)),
            pl.BlockSpec((1, HID), lambda i: (0, 0)),
            pl.BlockSpec((HID, OUT_CH), lambda i: (0, 0)),
        ],
        out_specs=pl.BlockSpec((2, _TC_R, OUT_CH), lambda i: (0, i, 0)),
        out_shape=jax.ShapeDtypeStruct((2, N_PAD, OUT_CH), jnp.float32),
    )(agg1, hs1, dinv, b1, W2)


def _ep2_call(agg2, hs2d, dinv, b2):
    return pl.pallas_call(
        _ep2_body,
        grid=_GRID,
        in_specs=[
            pl.BlockSpec((2, _TC_R, OUT_CH), lambda i: (0, i, 0)),
            pl.BlockSpec((1, _TC_R, OUT_CH), lambda i: (0, i, 0)),
            pl.BlockSpec((_TC_R, 1), lambda i: (i, 0)),
            pl.BlockSpec((1, OUT_CH), lambda i: (0, 0)),
        ],
        out_specs=pl.BlockSpec((_TC_R, OUT_CH), lambda i: (i, 0)),
        out_shape=jax.ShapeDtypeStruct((N_PAD, OUT_CH), jnp.float32),
    )(agg2, hs2d, dinv, b2)


@jax.jit
def kernel(x, edge_index, W1, b1, W2, b2):
    src = edge_index[0].astype(jnp.int32)
    dst = edge_index[1].astype(jnp.int32)
    pad = E_PAD - N_EDGES
    fill_src = jnp.full((pad,), N_NODES, jnp.int32)
    fill_dst = N_NODES + (jnp.arange(pad, dtype=jnp.int32) % (N_PAD - N_NODES))
    src_p = jnp.concatenate([src, fill_src])
    dst_p = jnp.concatenate([dst, fill_dst])
    src3 = src_p.reshape(N_TILES, CHA_SPLIT, CHUNK_A)
    src4 = jnp.stack([src3, src3 + N_PAD])
    dst3 = dst_p.reshape(N_TILES, CHA_SPLIT, CHUNK_A)
    srcr = src_p.reshape(N_CORES * N_TILES, CHA_EDGE, CHUNK_A)
    srcr_off = jnp.concatenate([srcr[:N_TILES], srcr[N_TILES:] + N_PAD])
    dstr = dst_p.reshape(N_CORES * N_TILES, CHA_EDGE, CHUNK_A)
    dstr_deg = dst_p.reshape(N_CORES * N_TILES, CH_EDGE, CHUNK)
    x_pad = jnp.pad(x, ((0, N_PAD - N_NODES), (0, 0)))
    zeros_nd = jnp.zeros((N_PAD, OUT_CH), jnp.float32)
    ones_c1 = jnp.ones((CHUNK, DEG_W), jnp.float32)

    degp = _deg_call(dstr_deg, ones_c1, zeros_nd)
    degp_col = degp[:, :, :1]
    hs1, dinv = _mm1_call(x_pad, W1, degp_col)
    agg1 = _agg_feat_call(hs1.reshape(2 * N_PAD, OUT_CH), src4, dst3,
                          zeros_nd)
    hs2d = _mm2_call(agg1, hs1, dinv, b1.reshape(1, HID), W2)
    agg2 = _agg_edge_call(hs2d.reshape(2 * N_PAD, OUT_CH), srcr_off, dstr,
                          zeros_nd)
    out = _ep2_call(agg2, hs2d, dinv, b2.reshape(1, OUT_CH))
    return out[:N_NODES]


# speedup vs baseline: 1.1564x; 1.0317x over previous
import functools

import jax
import jax.numpy as jnp
from jax import lax
from jax.experimental import pallas as pl
from jax.experimental.pallas import tpu as pltpu
from jax.experimental.pallas import tpu_sc as plsc

N_NODES = 10000
N_EDGES = 320000
IN_CH = 128
OUT_CH = 128
HID = 256

N_TILES = 16
N_CORES = 2
CHUNK = 128
N_PAD = 10112
E_PAD = 323584
ROWS_PER_TILE = N_PAD // N_TILES


def _sc_mesh():
    return plsc.VectorSubcoreMesh(core_axis_name="c", subcore_axis_name="s")


DEG_W = 128
CH_EDGE = E_PAD // (N_CORES * N_TILES) // CHUNK
CHUNK_A = 64
CHA_SPLIT = E_PAD // N_TILES // CHUNK_A
CHA_EDGE = E_PAD // (N_CORES * N_TILES) // CHUNK_A


def _deg_body(dst_h, ones_h, zeros_h, out_h, dst_v, ones_v, deg_sp, ssem):
    c = lax.axis_index("c")
    s = lax.axis_index("s")
    r0 = s * ROWS_PER_TILE
    pltpu.sync_copy(zeros_h.at[pl.ds(r0, ROWS_PER_TILE)],
                    deg_sp.at[pl.ds(r0, ROWS_PER_TILE)])
    pltpu.sync_copy(ones_h, ones_v)
    pltpu.sync_copy(dst_h.at[c * N_TILES + s], dst_v)
    plsc.subcore_barrier()

    def scat(i, b):
        return pltpu.make_async_copy(ones_v, deg_sp.at[dst_v.at[i]],
                                     ssem.at[b])

    def chunk(i, carry):
        b = lax.rem(i, 2)

        @pl.when(i >= 2)
        def _():
            scat(i - 2, b).wait()

        scat(i, b).start(add=True)
        return carry

    lax.fori_loop(0, CH_EDGE, chunk, 0)
    scat(CH_EDGE - 2, lax.rem(CH_EDGE - 2, 2)).wait()
    scat(CH_EDGE - 1, lax.rem(CH_EDGE - 1, 2)).wait()
    plsc.subcore_barrier()
    pltpu.sync_copy(deg_sp.at[pl.ds(r0, ROWS_PER_TILE)],
                    out_h.at[c, pl.ds(r0, ROWS_PER_TILE)])


_deg_call = pl.kernel(
    _deg_body,
    out_type=jax.ShapeDtypeStruct((N_CORES, N_PAD, DEG_W), jnp.float32),
    mesh=_sc_mesh(),
    scratch_types=[
        pltpu.VMEM((CH_EDGE, CHUNK), jnp.int32),
        pltpu.VMEM((CHUNK, DEG_W), jnp.float32),
        pltpu.VMEM_SHARED((N_PAD, DEG_W), jnp.float32),
        pltpu.SemaphoreType.DMA((2,)),
    ],
)


def _make_agg(feature_split):
    NR = 5
    G = 3
    NI = 8
    CH0 = 220
    CH1 = 2 * CHA_EDGE - CH0

    def body(table_h, src_h, dst_h, zeros_h, out_h, idx_s, idx_d, rows_v,
             agg_sp, isem, dsem, gsem, ssem):
        c = lax.axis_index("c")
        s = lax.axis_index("s")
        r0 = s * ROWS_PER_TILE
        pltpu.sync_copy(zeros_h.at[pl.ds(r0, ROWS_PER_TILE)],
                        agg_sp.at[pl.ds(r0, ROWS_PER_TILE)])
        plsc.subcore_barrier()

        def pipeline(src_row, dst_row, n_chunks):
            def isrc(i):
                b = lax.rem(i, NI)
                return pltpu.make_async_copy(src_row(i), idx_s.at[b],
                                             isem.at[b])

            def idst(i):
                b = lax.rem(i, NI)
                return pltpu.make_async_copy(dst_row(i), idx_d.at[b],
                                             dsem.at[b])

            def gath(i):
                b = lax.rem(i, NR)
                return pltpu.make_async_copy(
                    table_h.at[idx_s.at[lax.rem(i, NI)]], rows_v.at[b],
                    gsem.at[b])

            def scat(i):
                b = lax.rem(i, NR)
                return pltpu.make_async_copy(
                    rows_v.at[b], agg_sp.at[idx_d.at[lax.rem(i, NI)]],
                    ssem.at[b])

            for k in range(G + 1):
                isrc(k).start()
                idst(k).start()
            for k in range(G):
                isrc(k).wait()
                idst(k).wait()
                gath(k).start()

            def chunk(i, carry):
                @pl.when(i >= NR - G)
                def _():
                    scat(i - (NR - G)).wait()

                gath(i).wait()
                scat(i).start(add=True)

                @pl.when(i + G < n_chunks)
                def _():
                    isrc(i + G).wait()
                    idst(i + G).wait()
                    gath(i + G).start()

                @pl.when(i + G + 1 < n_chunks)
                def _():
                    isrc(i + G + 1).start()
                    idst(i + G + 1).start()

                return carry

            lax.fori_loop(0, n_chunks, chunk, 0)
            for k in range(NR - G):
                scat(n_chunks - (NR - G) + k).wait()

        if feature_split:
            pipeline(lambda i: src_h.at[c, s, i], lambda i: dst_h.at[s, i],
                     CHA_SPLIT)
        else:
            @pl.when(c == 0)
            def _():
                pipeline(lambda i: src_h.at[c, s * CH0 + i],
                         lambda i: dst_h.at[s * CH0 + i], CH0)

            @pl.when(c == 1)
            def _():
                base = N_TILES * CH0 + s * CH1
                pipeline(lambda i: src_h.at[c, base + i],
                         lambda i: dst_h.at[base + i], CH1)

        plsc.subcore_barrier()
        pltpu.sync_copy(agg_sp.at[pl.ds(r0, ROWS_PER_TILE)],
                        out_h.at[c, pl.ds(r0, ROWS_PER_TILE)])

    return pl.kernel(
        body,
        out_type=jax.ShapeDtypeStruct((N_CORES, N_PAD, OUT_CH), jnp.float32),
        mesh=_sc_mesh(),
        scratch_types=[
            pltpu.VMEM((NI, CHUNK_A), jnp.int32),
            pltpu.VMEM((NI, CHUNK_A), jnp.int32),
            pltpu.VMEM((NR, CHUNK_A, OUT_CH), jnp.float32),
            pltpu.VMEM_SHARED((N_PAD, OUT_CH), jnp.float32),
            pltpu.SemaphoreType.DMA((NI,)),
            pltpu.SemaphoreType.DMA((NI,)),
            pltpu.SemaphoreType.DMA((NR,)),
            pltpu.SemaphoreType.DMA((NR,)),
        ],
    )


_agg_feat_call = _make_agg(feature_split=True)
_agg_edge_call = _make_agg(feature_split=False)


def _mm1_body(x_ref, w_ref, degp_ref, hs_ref, dinv_ref):
    deg = degp_ref[0] + degp_ref[1] + 1.0
    dinv = jax.lax.rsqrt(deg)
    h = jnp.dot(x_ref[...], w_ref[...], preferred_element_type=jnp.float32)
    hs = h * dinv
    hs_ref[0] = hs[:, :OUT_CH]
    hs_ref[1] = hs[:, OUT_CH:]
    dinv_ref[...] = dinv


def _mm2_body(agg_ref, hs1_ref, dinv_ref, b1_ref, w2_ref, hs2_ref):
    dinv = dinv_ref[...]
    xa = jnp.maximum((agg_ref[0] + hs1_ref[0]) * dinv + b1_ref[:, :OUT_CH], 0.0)
    xb = jnp.maximum((agg_ref[1] + hs1_ref[1]) * dinv + b1_ref[:, OUT_CH:], 0.0)
    w2 = w2_ref[...]
    h2 = (jnp.dot(xa, w2[:OUT_CH], preferred_element_type=jnp.float32)
          + jnp.dot(xb, w2[OUT_CH:], preferred_element_type=jnp.float32))
    hs2 = h2 * dinv
    hs2_ref[0] = hs2
    hs2_ref[1] = hs2


def _ep2_body(agg_ref, hs2_ref, dinv_ref, b2_ref, o_ref):
    acc = agg_ref[0] + agg_ref[1] + hs2_ref[0]
    o_ref[...] = jnp.maximum(acc * dinv_ref[...] + b2_ref[...], 0.0)


_TC_R = 1264
_GRID = (N_PAD // _TC_R,)


def _mm1_call(x_pad, W1, degp_col):
    return pl.pallas_call(
        _mm1_body,
        grid=_GRID,
        in_specs=[
            pl.BlockSpec((_TC_R, IN_CH), lambda i: (i, 0)),
            pl.BlockSpec((IN_CH, HID), lambda i: (0, 0)),
            pl.BlockSpec((2, _TC_R, 1), lambda i: (0, i, 0)),
        ],
        out_specs=[
            pl.BlockSpec((2, _TC_R, OUT_CH), lambda i: (0, i, 0)),
            pl.BlockSpec((_TC_R, 1), lambda i: (i, 0)),
        ],
        out_shape=[
            jax.ShapeDtypeStruct((2, N_PAD, OUT_CH), jnp.float32),
            jax.ShapeDtypeStruct((N_PAD, 1), jnp.float32),
        ],
    )(x_pad, W1, degp_col)


def _mm2_call(agg1, hs1, dinv, b1, W2):
    return pl.pallas_call(
        _mm2_body,
        grid=_GRID,
        in_specs=[
            pl.BlockSpec((2, _TC_R, OUT_CH), lambda i: (0, i, 0)),
            pl.BlockSpec((2, _TC_R, OUT_CH), lambda i: (0, i, 0)),
            pl.BlockSpec((_TC_R, 1), lambda i: (i, 0)),
            pl.BlockSpec((1, HID), lambda i: (0, 0)),
            pl.BlockSpec((HID, OUT_CH), lambda i: (0, 0)),
        ],
        out_specs=pl.BlockSpec((2, _TC_R, OUT_CH), lambda i: (0, i, 0)),
        out_shape=jax.ShapeDtypeStruct((2, N_PAD, OUT_CH), jnp.float32),
    )(agg1, hs1, dinv, b1, W2)


def _ep2_call(agg2, hs2d, dinv, b2):
    return pl.pallas_call(
        _ep2_body,
        grid=_GRID,
        in_specs=[
            pl.BlockSpec((2, _TC_R, OUT_CH), lambda i: (0, i, 0)),
            pl.BlockSpec((1, _TC_R, OUT_CH), lambda i: (0, i, 0)),
            pl.BlockSpec((_TC_R, 1), lambda i: (i, 0)),
            pl.BlockSpec((1, OUT_CH), lambda i: (0, 0)),
        ],
        out_specs=pl.BlockSpec((_TC_R, OUT_CH), lambda i: (i, 0)),
        out_shape=jax.ShapeDtypeStruct((N_PAD, OUT_CH), jnp.float32),
    )(agg2, hs2d, dinv, b2)


@jax.jit
def kernel(x, edge_index, W1, b1, W2, b2):
    src = edge_index[0].astype(jnp.int32)
    dst = edge_index[1].astype(jnp.int32)
    pad = E_PAD - N_EDGES
    fill_src = jnp.full((pad,), N_NODES, jnp.int32)
    fill_dst = N_NODES + (jnp.arange(pad, dtype=jnp.int32) % (N_PAD - N_NODES))
    src_p = jnp.concatenate([src, fill_src])
    dst_p = jnp.concatenate([dst, fill_dst])
    src3 = src_p.reshape(N_TILES, CHA_SPLIT, CHUNK_A)
    src4 = jnp.stack([src3, src3 + N_PAD])
    dst3 = dst_p.reshape(N_TILES, CHA_SPLIT, CHUNK_A)
    srcg = src_p.reshape(E_PAD // CHUNK_A, CHUNK_A)
    srcg2 = jnp.stack([srcg, srcg + N_PAD])
    dstg = dst_p.reshape(E_PAD // CHUNK_A, CHUNK_A)
    dstr_deg = dst_p.reshape(N_CORES * N_TILES, CH_EDGE, CHUNK)
    x_pad = jnp.pad(x, ((0, N_PAD - N_NODES), (0, 0)))
    zeros_nd = jnp.zeros((N_PAD, OUT_CH), jnp.float32)
    ones_c1 = jnp.ones((CHUNK, DEG_W), jnp.float32)

    degp = _deg_call(dstr_deg, ones_c1, zeros_nd)
    degp_col = degp[:, :, :1]
    hs1, dinv = _mm1_call(x_pad, W1, degp_col)
    agg1 = _agg_feat_call(hs1.reshape(2 * N_PAD, OUT_CH), src4, dst3,
                          zeros_nd)
    hs2d = _mm2_call(agg1, hs1, dinv, b1.reshape(1, HID), W2)
    agg2 = _agg_edge_call(hs2d.reshape(2 * N_PAD, OUT_CH), srcg2, dstg,
                          zeros_nd)
    out = _ep2_call(agg2, hs2d, dinv, b2.reshape(1, OUT_CH))
    return out[:N_NODES]
